# baseline (speedup 1.0000x reference)
import functools

import jax
import jax.numpy as jnp
from jax import lax
from jax.experimental import pallas as pl
from jax.experimental.pallas import tpu as pltpu
from jax.experimental.pallas import tpu_sc as plsc

N = 10000
E = 320000
F = 128

NC = 2
NS = 16
NW = NC * NS
K = 112
NCH = 90
CPW = 93
SLABS = CPW + 3
EPWP = NCH * K
NPAD = 10112
RPS = NPAD // NS

_mesh = plsc.VectorSubcoreMesh(core_axis_name="c", subcore_axis_name="s")


@functools.partial(
    pl.kernel,
    out_type=jax.ShapeDtypeStruct((NC, NPAD, F), jnp.float32),
    mesh=_mesh,
    scratch_types=[
        pltpu.VMEM((3, 4, 128), jnp.int32),
        pltpu.VMEM((3 * 128,), jnp.float32),
        pltpu.VMEM((3, K), jnp.int32),
        pltpu.VMEM((3, K, F), jnp.float32),
        pltpu.VMEM_SHARED((NPAD, F), jnp.float32),
        pltpu.SemaphoreType.DMA,
        pltpu.SemaphoreType.DMA,
        pltpu.SemaphoreType.DMA,
        pltpu.SemaphoreType.DMA,
        pltpu.SemaphoreType.DMA,
        pltpu.SemaphoreType.DMA,
        pltpu.SemaphoreType.DMA,
        pltpu.SemaphoreType.DMA,
        pltpu.SemaphoreType.DMA,
    ],
    compiler_params=pltpu.CompilerParams(needs_layout_passes=False),
)
def _spmm_sc(sup_hbm, pk_hbm, zer_hbm, out_hbm, pk_v, w_v, dsti_v,
             rows_v, acc, pk_sem0, pk_sem1, pk_sem2, g_sem0, g_sem1, g_sem2,
             s_sem0, s_sem1, s_sem2):
    pk_sem = (pk_sem0, pk_sem1, pk_sem2)
    g_sem = (g_sem0, g_sem1, g_sem2)
    s_sem = (s_sem0, s_sem1, s_sem2)
    c = lax.axis_index("c")
    s = lax.axis_index("s")
    wid = c * NS + s
    sbase = wid * SLABS


    def start_pk(j, b):
        pltpu.async_copy(pk_hbm.at[sbase + j], pk_v.at[b], pk_sem[b])

    def wait_pk(b):
        pltpu.make_async_copy(pk_hbm.at[0], pk_v.at[b], pk_sem[b]).wait()

    def start_gather(b):
        pltpu.async_copy(sup_hbm.at[pk_v.at[b, 0, pl.ds(0, K)]],
                         rows_v.at[b], g_sem[b])

    def wait_gather(b):
        pltpu.make_async_copy(sup_hbm.at[pk_v.at[b, 0, pl.ds(0, K)]],
                              rows_v.at[b], g_sem[b]).wait()

    def start_scatter(b):
        pltpu.async_copy(rows_v.at[b], acc.at[dsti_v.at[b]], s_sem[b],
                         add=True)

    def wait_scatter(b):
        pltpu.make_async_copy(rows_v.at[b], acc.at[dsti_v.at[b]],
                              s_sem[b]).wait()

    def copy_dst(b):
        for i in range(K // 16):
            dsti_v[b, pl.ds(i * 16, 16)] = pk_v[b, 1, pl.ds(i * 16, 16)]

    def copy_w(b):
        for i in range(K // 16):
            w_v[pl.ds(b * 128 + i * 16, 16)] = plsc.bitcast(
                pk_v[b, 2, pl.ds(i * 16, 16)], jnp.float32)

    def scale_rows(b):
        def body(i, carry):
            w = plsc.load_gather(w_v,
                                 [jnp.full((16,), b * 128, jnp.int32) + i])
            for f in range(F // 16):
                rows_v[b, i, pl.ds(f * 16, 16)] = (
                    rows_v[b, i, pl.ds(f * 16, 16)] * w)
            return carry

        lax.fori_loop(0, K, body, 0, unroll=2)

    def process(j, b, first=False):
        nb = (b + 1) % 3
        wait_gather(b)
        if not first:
            wait_scatter(nb)
        wait_pk(nb)
        start_gather(nb)
        copy_dst(b)
        copy_w(b)
        start_pk(j + 3, b)
        scale_rows(b)
        start_scatter(b)

    pltpu.sync_copy(zer_hbm, acc.at[pl.ds(s * RPS, RPS)])
    plsc.subcore_barrier()

    start_pk(0, 0)
    start_pk(1, 1)
    start_pk(2, 2)
    wait_pk(0)
    start_gather(0)

    process(0, 0, first=True)
    process(1, 1, first=True)
    process(2, 2)

    def outer(g, carry):
        process(3 * g, 0)
        process(3 * g + 1, 1)
        process(3 * g + 2, 2)
        return carry

    lax.fori_loop(1, CPW // 3, outer, 0)

    wait_gather(CPW % 3)
    wait_pk((CPW + 1) % 3)
    wait_pk((CPW + 2) % 3)
    wait_scatter((CPW - 2) % 3)
    wait_scatter((CPW - 1) % 3)
    plsc.subcore_barrier()
    pltpu.sync_copy(acc.at[pl.ds(s * RPS, RPS)],
                    out_hbm.at[c].at[pl.ds(s * RPS, RPS)])


def _pack_edges(src, dst, w):
    tot = NW * CPW * K
    pad = tot - E
    fill = (jnp.arange(pad, dtype=jnp.int32) * 97) % N
    srcp = jnp.concatenate([src, fill]).reshape(NW, CPW, K)
    dstp = jnp.concatenate([dst, fill]).reshape(NW, CPW, K)
    wbits = lax.bitcast_convert_type(
        jnp.concatenate([w, jnp.zeros((pad,), w.dtype)]),
        jnp.int32).reshape(NW, CPW, K)
    pk = jnp.stack([srcp, dstp, wbits, jnp.zeros_like(srcp)], axis=2)
    pk = jnp.pad(pk, ((0, 0), (0, SLABS - CPW), (0, 0), (0, 128 - K)))
    return pk.reshape(NW * SLABS, 4, 128)


def _mm_body(x_ref, w_ref, o_ref):
    o_ref[...] = jnp.dot(x_ref[...], w_ref[...],
                         preferred_element_type=jnp.float32)


def _mm(x, W, bm=1000):
    m = x.shape[0]
    return pl.pallas_call(
        _mm_body,
        grid=(m // bm,),
        in_specs=[pl.BlockSpec((bm, F), lambda i: (i, 0)),
                  pl.BlockSpec((F, F), lambda i: (0, 0))],
        out_specs=pl.BlockSpec((bm, F), lambda i: (i, 0)),
        out_shape=jax.ShapeDtypeStruct((m, F), jnp.float32),
    )(x, W)


def _mid_body(p_ref, b_ref, w_ref, o_ref):
    h = jnp.maximum(p_ref[0] + p_ref[1] + b_ref[...], 0.0)
    o_ref[...] = jnp.dot(h, w_ref[...], preferred_element_type=jnp.float32)


def _mid(p, b1, W2, bm=1000):
    return pl.pallas_call(
        _mid_body,
        grid=(N // bm,),
        in_specs=[pl.BlockSpec((NC, bm, F), lambda i: (0, i, 0)),
                  pl.BlockSpec((1, F), lambda i: (0, 0)),
                  pl.BlockSpec((F, F), lambda i: (0, 0))],
        out_specs=pl.BlockSpec((bm, F), lambda i: (i, 0)),
        out_shape=jax.ShapeDtypeStruct((N, F), jnp.float32),
    )(p, b1.reshape(1, F), W2)


def _fin_body(p_ref, b_ref, o_ref):
    o_ref[...] = p_ref[0] + p_ref[1] + b_ref[...]


def _fin(p, b2, bm=1000):
    return pl.pallas_call(
        _fin_body,
        grid=(N // bm,),
        in_specs=[pl.BlockSpec((NC, bm, F), lambda i: (0, i, 0)),
                  pl.BlockSpec((1, F), lambda i: (0, 0))],
        out_specs=pl.BlockSpec((bm, F), lambda i: (i, 0)),
        out_shape=jax.ShapeDtypeStruct((N, F), jnp.float32),
    )(p, b2.reshape(1, F))


def kernel(x, edge_index, edge_weight, W1, b1, W2, b2):
    pk = _pack_edges(edge_index[0], edge_index[1], edge_weight)
    zer = jnp.zeros((RPS, F), dtype=jnp.float32)

    support1 = _mm(x, W1)
    p1 = _spmm_sc(support1, pk, zer)
    support2 = _mid(p1, b1, W2)
    p2 = _spmm_sc(support2, pk, zer)
    return _fin(p2, b2)


# speedup vs baseline: 1.0471x; 1.0471x over previous
import functools

import jax
import jax.numpy as jnp
from jax import lax
from jax.experimental import pallas as pl
from jax.experimental.pallas import tpu as pltpu
from jax.experimental.pallas import tpu_sc as plsc

N = 10000
E = 320000
F = 128

NC = 2
NS = 16
NW = NC * NS
K = 96
NCH = 107
CPW = 108
SLABS = CPW + 3
EPWP = NCH * K
NPAD = 10112
RPS = NPAD // NS

_mesh = plsc.VectorSubcoreMesh(core_axis_name="c", subcore_axis_name="s")


@functools.partial(
    pl.kernel,
    out_type=jax.ShapeDtypeStruct((NC, NPAD, F), jnp.float32),
    mesh=_mesh,
    scratch_types=[
        pltpu.VMEM((3, 4, 128), jnp.int32),
        pltpu.VMEM((3 * 128,), jnp.float32),
        pltpu.VMEM((3, K), jnp.int32),
        pltpu.VMEM((3, K, F), jnp.float32),
        pltpu.VMEM_SHARED((NPAD, F), jnp.float32),
        pltpu.SemaphoreType.DMA,
        pltpu.SemaphoreType.DMA,
        pltpu.SemaphoreType.DMA,
        pltpu.SemaphoreType.DMA,
        pltpu.SemaphoreType.DMA,
        pltpu.SemaphoreType.DMA,
        pltpu.SemaphoreType.DMA,
        pltpu.SemaphoreType.DMA,
        pltpu.SemaphoreType.DMA,
    ],
    compiler_params=pltpu.CompilerParams(needs_layout_passes=False),
)
def _spmm_sc(sup_hbm, pk_hbm, zer_hbm, out_hbm, pk_v, w_v, dsti_v,
             rows_v, acc, pk_sem0, pk_sem1, pk_sem2, g_sem0, g_sem1, g_sem2,
             s_sem0, s_sem1, s_sem2):
    pk_sem = (pk_sem0, pk_sem1, pk_sem2)
    g_sem = (g_sem0, g_sem1, g_sem2)
    s_sem = (s_sem0, s_sem1, s_sem2)
    c = lax.axis_index("c")
    s = lax.axis_index("s")
    wid = c * NS + s
    sbase = wid * SLABS


    def start_pk(j, b):
        pltpu.async_copy(pk_hbm.at[sbase + j], pk_v.at[b], pk_sem[b])

    def wait_pk(b):
        pltpu.make_async_copy(pk_hbm.at[0], pk_v.at[b], pk_sem[b]).wait()

    def start_gather(b):
        pltpu.async_copy(sup_hbm.at[pk_v.at[b, 0, pl.ds(0, K)]],
                         rows_v.at[b], g_sem[b])

    def wait_gather(b):
        pltpu.make_async_copy(sup_hbm.at[pk_v.at[b, 0, pl.ds(0, K)]],
                              rows_v.at[b], g_sem[b]).wait()

    def start_scatter(b):
        pltpu.async_copy(rows_v.at[b], acc.at[dsti_v.at[b]], s_sem[b],
                         add=True)

    def wait_scatter(b):
        pltpu.make_async_copy(rows_v.at[b], acc.at[dsti_v.at[b]],
                              s_sem[b]).wait()

    def copy_dst(b):
        for i in range(K // 16):
            dsti_v[b, pl.ds(i * 16, 16)] = pk_v[b, 1, pl.ds(i * 16, 16)]

    def copy_w(b):
        for i in range(K // 16):
            w_v[pl.ds(b * 128 + i * 16, 16)] = plsc.bitcast(
                pk_v[b, 2, pl.ds(i * 16, 16)], jnp.float32)

    def scale_rows(b):
        def body(i, carry):
            w = plsc.load_gather(w_v,
                                 [jnp.full((16,), b * 128, jnp.int32) + i])
            for f in range(F // 16):
                rows_v[b, i, pl.ds(f * 16, 16)] = (
                    rows_v[b, i, pl.ds(f * 16, 16)] * w)
            return carry

        lax.fori_loop(0, K, body, 0, unroll=2)

    def process(j, b, first=False):
        nb = (b + 1) % 3
        wait_gather(b)
        if not first:
            wait_scatter(nb)
        wait_pk(nb)
        start_gather(nb)
        copy_dst(b)
        copy_w(b)
        start_pk(j + 3, b)
        scale_rows(b)
        start_scatter(b)

    pltpu.sync_copy(zer_hbm, acc.at[pl.ds(s * RPS, RPS)])
    plsc.subcore_barrier()

    start_pk(0, 0)
    start_pk(1, 1)
    start_pk(2, 2)
    wait_pk(0)
    start_gather(0)

    process(0, 0, first=True)
    process(1, 1, first=True)
    process(2, 2)

    def outer(g, carry):
        process(3 * g, 0)
        process(3 * g + 1, 1)
        process(3 * g + 2, 2)
        return carry

    lax.fori_loop(1, CPW // 3, outer, 0)

    wait_gather(CPW % 3)
    wait_pk((CPW + 1) % 3)
    wait_pk((CPW + 2) % 3)
    wait_scatter((CPW - 2) % 3)
    wait_scatter((CPW - 1) % 3)
    plsc.subcore_barrier()
    pltpu.sync_copy(acc.at[pl.ds(s * RPS, RPS)],
                    out_hbm.at[c].at[pl.ds(s * RPS, RPS)])


def _pack_edges(src, dst, w):
    tot = NW * CPW * K
    pad = tot - E
    fill = (jnp.arange(pad, dtype=jnp.int32) * 97) % N
    srcp = jnp.concatenate([src, fill]).reshape(NW, CPW, K)
    dstp = jnp.concatenate([dst, fill]).reshape(NW, CPW, K)
    wbits = lax.bitcast_convert_type(
        jnp.concatenate([w, jnp.zeros((pad,), w.dtype)]),
        jnp.int32).reshape(NW, CPW, K)
    pk = jnp.stack([srcp, dstp, wbits, jnp.zeros_like(srcp)], axis=2)
    pk = jnp.pad(pk, ((0, 0), (0, SLABS - CPW), (0, 0), (0, 128 - K)))
    return pk.reshape(NW * SLABS, 4, 128)


def _mm_body(x_ref, w_ref, o_ref):
    o_ref[...] = jnp.dot(x_ref[...], w_ref[...],
                         preferred_element_type=jnp.float32)


def _mm(x, W, bm=1000):
    m = x.shape[0]
    return pl.pallas_call(
        _mm_body,
        grid=(m // bm,),
        in_specs=[pl.BlockSpec((bm, F), lambda i: (i, 0)),
                  pl.BlockSpec((F, F), lambda i: (0, 0))],
        out_specs=pl.BlockSpec((bm, F), lambda i: (i, 0)),
        out_shape=jax.ShapeDtypeStruct((m, F), jnp.float32),
    )(x, W)


def _mid_body(p_ref, b_ref, w_ref, o_ref):
    h = jnp.maximum(p_ref[0] + p_ref[1] + b_ref[...], 0.0)
    o_ref[...] = jnp.dot(h, w_ref[...], preferred_element_type=jnp.float32)


def _mid(p, b1, W2, bm=1000):
    return pl.pallas_call(
        _mid_body,
        grid=(N // bm,),
        in_specs=[pl.BlockSpec((NC, bm, F), lambda i: (0, i, 0)),
                  pl.BlockSpec((1, F), lambda i: (0, 0)),
                  pl.BlockSpec((F, F), lambda i: (0, 0))],
        out_specs=pl.BlockSpec((bm, F), lambda i: (i, 0)),
        out_shape=jax.ShapeDtypeStruct((N, F), jnp.float32),
    )(p, b1.reshape(1, F), W2)


def _fin_body(p_ref, b_ref, o_ref):
    o_ref[...] = p_ref[0] + p_ref[1] + b_ref[...]


def _fin(p, b2, bm=1000):
    return pl.pallas_call(
        _fin_body,
        grid=(N // bm,),
        in_specs=[pl.BlockSpec((NC, bm, F), lambda i: (0, i, 0)),
                  pl.BlockSpec((1, F), lambda i: (0, 0))],
        out_specs=pl.BlockSpec((bm, F), lambda i: (i, 0)),
        out_shape=jax.ShapeDtypeStruct((N, F), jnp.float32),
    )(p, b2.reshape(1, F))


def kernel(x, edge_index, edge_weight, W1, b1, W2, b2):
    pk = _pack_edges(edge_index[0], edge_index[1], edge_weight)
    zer = jnp.zeros((RPS, F), dtype=jnp.float32)

    support1 = _mm(x, W1)
    p1 = _spmm_sc(support1, pk, zer)
    support2 = _mid(p1, b1, W2)
    p2 = _spmm_sc(support2, pk, zer)
    return _fin(p2, b2)


# speedup vs baseline: 1.1006x; 1.0511x over previous
import functools

import jax
import jax.numpy as jnp
from jax import lax
from jax.experimental import pallas as pl
from jax.experimental.pallas import tpu as pltpu
from jax.experimental.pallas import tpu_sc as plsc

N = 10000
E = 320000
F = 128

NC = 2
NS = 16
NW = NC * NS
K = 96
NCH = 107
CPW = 108
SLABS = CPW + 3
EPWP = NCH * K
NPAD = 10112
RPS = NPAD // NS

_mesh = plsc.VectorSubcoreMesh(core_axis_name="c", subcore_axis_name="s")


@functools.partial(
    pl.kernel,
    out_type=jax.ShapeDtypeStruct((NC, NPAD, F), jnp.float32),
    mesh=_mesh,
    scratch_types=[
        pltpu.VMEM((3, 2, 128), jnp.int32),
        pltpu.VMEM((3 * 128,), jnp.float32),
        pltpu.VMEM((3, K), jnp.int32),
        pltpu.VMEM((3, K, F), jnp.float32),
        pltpu.VMEM_SHARED((NPAD, F), jnp.float32),
        pltpu.SemaphoreType.DMA,
        pltpu.SemaphoreType.DMA,
        pltpu.SemaphoreType.DMA,
        pltpu.SemaphoreType.DMA,
        pltpu.SemaphoreType.DMA,
        pltpu.SemaphoreType.DMA,
        pltpu.SemaphoreType.DMA,
        pltpu.SemaphoreType.DMA,
        pltpu.SemaphoreType.DMA,
    ],
    compiler_params=pltpu.CompilerParams(needs_layout_passes=False),
)
def _spmm_sc(sup_hbm, pk_hbm, w_hbm, zer_hbm, out_hbm, pk_v, w_v, dsti_v,
             rows_v, acc, pk_sem0, pk_sem1, pk_sem2, g_sem0, g_sem1, g_sem2,
             s_sem0, s_sem1, s_sem2):
    pk_sem = (pk_sem0, pk_sem1, pk_sem2)
    g_sem = (g_sem0, g_sem1, g_sem2)
    s_sem = (s_sem0, s_sem1, s_sem2)
    c = lax.axis_index("c")
    s = lax.axis_index("s")
    wid = c * NS + s
    sbase = wid * SLABS


    def start_pk(j, b):
        pltpu.async_copy(pk_hbm.at[sbase + j], pk_v.at[b], pk_sem[b])
        pltpu.async_copy(w_hbm.at[sbase + j], w_v.at[pl.ds(b * 128, 128)],
                         pk_sem[b])

    def wait_pk(b):
        pltpu.make_async_copy(pk_hbm.at[0], pk_v.at[b], pk_sem[b]).wait()
        pltpu.make_async_copy(w_hbm.at[0], w_v.at[pl.ds(b * 128, 128)],
                              pk_sem[b]).wait()

    def start_gather(b):
        pltpu.async_copy(sup_hbm.at[pk_v.at[b, 0, pl.ds(0, K)]],
                         rows_v.at[b], g_sem[b])

    def wait_gather(b):
        pltpu.make_async_copy(sup_hbm.at[pk_v.at[b, 0, pl.ds(0, K)]],
                              rows_v.at[b], g_sem[b]).wait()

    def start_scatter(b):
        pltpu.async_copy(rows_v.at[b], acc.at[dsti_v.at[b]], s_sem[b],
                         add=True)

    def wait_scatter(b):
        pltpu.make_async_copy(rows_v.at[b], acc.at[dsti_v.at[b]],
                              s_sem[b]).wait()

    def copy_dst(b):
        for i in range(K // 16):
            dsti_v[b, pl.ds(i * 16, 16)] = pk_v[b, 1, pl.ds(i * 16, 16)]

    def scale_rows(b):
        def body(i, carry):
            w = plsc.load_gather(w_v,
                                 [jnp.full((16,), b * 128, jnp.int32) + i])
            for f in range(F // 16):
                rows_v[b, i, pl.ds(f * 16, 16)] = (
                    rows_v[b, i, pl.ds(f * 16, 16)] * w)
            return carry

        lax.fori_loop(0, K, body, 0, unroll=2)

    def process(j, b, first=False):
        nb = (b + 1) % 3
        if not first:
            wait_scatter(nb)
        wait_pk(nb)
        start_gather(nb)
        wait_gather(b)
        copy_dst(b)
        scale_rows(b)
        start_pk(j + 3, b)
        start_scatter(b)

    pltpu.sync_copy(zer_hbm, acc.at[pl.ds(s * RPS, RPS)])
    plsc.subcore_barrier()

    start_pk(0, 0)
    start_pk(1, 1)
    start_pk(2, 2)
    wait_pk(0)
    start_gather(0)

    process(0, 0, first=True)
    process(1, 1, first=True)
    process(2, 2)

    def outer(g, carry):
        process(3 * g, 0)
        process(3 * g + 1, 1)
        process(3 * g + 2, 2)
        return carry

    lax.fori_loop(1, CPW // 3, outer, 0)

    wait_gather(CPW % 3)
    wait_pk((CPW + 1) % 3)
    wait_pk((CPW + 2) % 3)
    wait_scatter((CPW - 2) % 3)
    wait_scatter((CPW - 1) % 3)
    plsc.subcore_barrier()
    pltpu.sync_copy(acc.at[pl.ds(s * RPS, RPS)],
                    out_hbm.at[c].at[pl.ds(s * RPS, RPS)])


def _pack_edges(src, dst, w):
    tot = NW * CPW * K
    pad = tot - E
    fill = (jnp.arange(pad, dtype=jnp.int32) * 97) % N
    srcp = jnp.concatenate([src, fill]).reshape(NW, CPW, K)
    dstp = jnp.concatenate([dst, fill]).reshape(NW, CPW, K)
    pk = jnp.stack([srcp, dstp], axis=2)
    pk = jnp.pad(pk, ((0, 0), (0, SLABS - CPW), (0, 0), (0, 128 - K)))
    wp = jnp.concatenate([w, jnp.zeros((pad,), w.dtype)]).reshape(NW, CPW, K)
    wp = jnp.pad(wp, ((0, 0), (0, SLABS - CPW), (0, 128 - K)))
    return pk.reshape(NW * SLABS, 2, 128), wp.reshape(NW * SLABS, 128)


def _mm_body(x_ref, w_ref, o_ref):
    o_ref[...] = jnp.dot(x_ref[...], w_ref[...],
                         preferred_element_type=jnp.float32)


def _mm(x, W, bm=1000):
    m = x.shape[0]
    return pl.pallas_call(
        _mm_body,
        grid=(m // bm,),
        in_specs=[pl.BlockSpec((bm, F), lambda i: (i, 0)),
                  pl.BlockSpec((F, F), lambda i: (0, 0))],
        out_specs=pl.BlockSpec((bm, F), lambda i: (i, 0)),
        out_shape=jax.ShapeDtypeStruct((m, F), jnp.float32),
    )(x, W)


def _mid_body(p_ref, b_ref, w_ref, o_ref):
    h = jnp.maximum(p_ref[0] + p_ref[1] + b_ref[...], 0.0)
    o_ref[...] = jnp.dot(h, w_ref[...], preferred_element_type=jnp.float32)


def _mid(p, b1, W2, bm=1000):
    return pl.pallas_call(
        _mid_body,
        grid=(N // bm,),
        in_specs=[pl.BlockSpec((NC, bm, F), lambda i: (0, i, 0)),
                  pl.BlockSpec((1, F), lambda i: (0, 0)),
                  pl.BlockSpec((F, F), lambda i: (0, 0))],
        out_specs=pl.BlockSpec((bm, F), lambda i: (i, 0)),
        out_shape=jax.ShapeDtypeStruct((N, F), jnp.float32),
    )(p, b1.reshape(1, F), W2)


def _fin_body(p_ref, b_ref, o_ref):
    o_ref[...] = p_ref[0] + p_ref[1] + b_ref[...]


def _fin(p, b2, bm=1000):
    return pl.pallas_call(
        _fin_body,
        grid=(N // bm,),
        in_specs=[pl.BlockSpec((NC, bm, F), lambda i: (0, i, 0)),
                  pl.BlockSpec((1, F), lambda i: (0, 0))],
        out_specs=pl.BlockSpec((bm, F), lambda i: (i, 0)),
        out_shape=jax.ShapeDtypeStruct((N, F), jnp.float32),
    )(p, b2.reshape(1, F))


def kernel(x, edge_index, edge_weight, W1, b1, W2, b2):
    pk, pw = _pack_edges(edge_index[0], edge_index[1], edge_weight)
    zer = jnp.zeros((RPS, F), dtype=jnp.float32)

    support1 = _mm(x, W1)
    p1 = _spmm_sc(support1, pk, pw, zer)
    support2 = _mid(p1, b1, W2)
    p2 = _spmm_sc(support2, pk, pw, zer)
    return _fin(p2, b2)
